# baseline (speedup 1.0000x reference)
import jax
import jax.numpy as jnp
import numpy as np
from jax.experimental import pallas as pl
from jax.experimental.pallas import tpu as pltpu

_K = 64
_P = 32768
_B = 64
_R = 8
_C = _P // _R
_EPS = float(np.finfo(np.float32).tiny)


def _pert_kernel(logits_ref, x_hbm_ref, pv_ref, out_ref, pert_ref,
                 x_vmem_ref, dma_sem):
    j = pl.program_id(0)

    @pl.when(j == 0)
    def _start_x_copy():
        pltpu.make_async_copy(x_hbm_ref, x_vmem_ref, dma_sem).start()

    @pl.when(j == 0)
    def _compute_pert():
        lg = logits_ref[...]
        u0 = jnp.exp(lg - jnp.max(lg, axis=(0, 1), keepdims=True))

        ones_w = jnp.ones((_C, 128), jnp.float32)

        def full_sum(v):
            t1 = jax.lax.dot_general(
                v, ones_w, (((1,), (0,)), ((), ())),
                precision=jax.lax.Precision.HIGHEST)
            return jnp.sum(t1[:, :1], axis=(0, 1), keepdims=True)

        def khot_body(_, carry):
            u, khot = carry
            s = full_sum(u)
            oa = u * (1.0 / s)
            khot = khot + oa
            u = u * jnp.maximum(1.0 - oa, _EPS)
            return (u, khot)

        zeros = jnp.zeros_like(u0)
        _, khot = jax.lax.fori_loop(0, _K, khot_body, (u0, zeros), unroll=16)

        kb = jax.lax.bitcast_convert_type(khot, jnp.int32)

        def count_ge(c):
            return jnp.sum(jnp.where(kb >= c, 1.0, 0.0),
                           axis=(0, 1), keepdims=True)

        def bit_body(i, t):
            b0 = jnp.int32(1) << (29 - 2 * i)
            b1 = b0 + b0
            c1, c2, c3 = t | b0, t | b1, t | (b1 | b0)
            n1, n2, n3 = count_ge(c1), count_ge(c2), count_ge(c3)
            return jnp.where(n3 >= 64.0, c3,
                             jnp.where(n2 >= 64.0, c2,
                                       jnp.where(n1 >= 64.0, c1, t)))

        t64 = jax.lax.fori_loop(0, 15, bit_body, jnp.zeros((1, 1), jnp.int32),
                                unroll=8)
        c_last = t64 | jnp.int32(1)
        t64 = jnp.where(count_ge(c_last) >= 64.0, c_last, t64)

        gt = kb > t64
        eq = kb == t64
        m = jnp.sum(jnp.where(gt, 1.0, 0.0), axis=(0, 1), keepdims=True)
        e = 64.0 - m
        idx = (jax.lax.broadcasted_iota(jnp.int32, (_R, _C), 0) * _C
               + jax.lax.broadcasted_iota(jnp.int32, (_R, _C), 1))

        def count_lt(c):
            return jnp.sum(jnp.where(eq & (idx < c), 1.0, 0.0),
                           axis=(0, 1), keepdims=True)

        def tie_body(i, t):
            b0 = jnp.int32(1) << (13 - 2 * i)
            b1 = b0 + b0
            c1, c2, c3 = t + b0, t + b1, t + b1 + b0
            n1, n2, n3 = count_lt(c1), count_lt(c2), count_lt(c3)
            return jnp.where(n3 < e, c3,
                             jnp.where(n2 < e, c2,
                                       jnp.where(n1 < e, c1, t)))

        i_star = jax.lax.fori_loop(0, 7, tie_body,
                                   jnp.zeros((1, 1), jnp.int32), unroll=7)
        i_last = i_star + jnp.int32(1)
        i_star = jnp.where(count_lt(i_last) < e, i_last, i_star)
        hard = gt | (eq & (idx <= i_star))
        pert_ref[...] = jnp.where(hard, (1.0 - khot) + khot, 0.0)

    @pl.when(j == 0)
    def _wait_x_copy():
        pltpu.make_async_copy(x_hbm_ref, x_vmem_ref, dma_sem).wait()

    x = x_vmem_ref[:, pl.ds(j * _C, _C)]
    pv = pv_ref[...]
    pert = pert_ref[pl.ds(j, 1), :]
    out_ref[...] = x + (pv - x) * pert


def kernel(x, logits, pert_val):
    return pl.pallas_call(
        _pert_kernel,
        grid=(_R,),
        in_specs=[
            pl.BlockSpec((_R, _C), lambda j: (0, 0)),
            pl.BlockSpec(memory_space=pl.ANY),
            pl.BlockSpec((1, _C), lambda j: (0, j)),
        ],
        out_specs=pl.BlockSpec((_B, _C), lambda j: (0, j)),
        out_shape=jax.ShapeDtypeStruct((_B, _P), jnp.float32),
        scratch_shapes=[pltpu.VMEM((_R, _C), jnp.float32),
                        pltpu.VMEM((_B, _P), jnp.float32),
                        pltpu.SemaphoreType.DMA],
        compiler_params=pltpu.CompilerParams(
            dimension_semantics=("arbitrary",)),
    )(logits.reshape(_R, _C), x, pert_val.reshape(1, _P))


# speedup vs baseline: 2.4724x; 2.4724x over previous
import jax
import jax.numpy as jnp
import numpy as np
from jax.experimental import pallas as pl
from jax.experimental.pallas import tpu as pltpu

_K = 64
_P = 32768
_B = 64
_R = 8
_C = _P // _R
_EPS = float(np.finfo(np.float32).tiny)


def _pert_kernel(logits_ref, x_hbm_ref, pv_ref, out_ref, pert_ref,
                 x_vmem_ref, dma_sem):
    j = pl.program_id(0)

    @pl.when(j == 0)
    def _start_x_copy():
        pltpu.make_async_copy(x_hbm_ref, x_vmem_ref, dma_sem).start()

    @pl.when(j == 0)
    def _compute_pert():
        lg = logits_ref[...]
        u0 = jnp.exp(lg - jnp.max(lg, axis=(0, 1), keepdims=True))

        def khot_body(_, carry):
            u, khot = carry
            s = jnp.sum(u, axis=(0, 1), keepdims=True)
            oa = u * (1.0 / s)
            khot = khot + oa
            u = u * jnp.maximum(1.0 - oa, _EPS)
            return (u, khot)

        zeros = jnp.zeros_like(u0)
        _, khot = jax.lax.fori_loop(0, _K, khot_body, (u0, zeros), unroll=16)

        kb = jax.lax.bitcast_convert_type(khot, jnp.int32)

        def count_ge(c):
            return jnp.sum(jnp.where(kb >= c, 1.0, 0.0),
                           axis=(0, 1), keepdims=True)

        def bit_body(i, t):
            b0 = jnp.int32(1) << (29 - 2 * i)
            b1 = b0 + b0
            c1, c2, c3 = t | b0, t | b1, t | (b1 | b0)
            n1, n2, n3 = count_ge(c1), count_ge(c2), count_ge(c3)
            return jnp.where(n3 >= 64.0, c3,
                             jnp.where(n2 >= 64.0, c2,
                                       jnp.where(n1 >= 64.0, c1, t)))

        t64 = jax.lax.fori_loop(0, 15, bit_body, jnp.zeros((1, 1), jnp.int32),
                                unroll=8)
        c_last = t64 | jnp.int32(1)
        t64 = jnp.where(count_ge(c_last) >= 64.0, c_last, t64)

        gt = kb > t64
        eq = kb == t64
        m = jnp.sum(jnp.where(gt, 1.0, 0.0), axis=(0, 1), keepdims=True)
        e = 64.0 - m
        idx = (jax.lax.broadcasted_iota(jnp.int32, (_R, _C), 0) * _C
               + jax.lax.broadcasted_iota(jnp.int32, (_R, _C), 1))

        def count_lt(c):
            return jnp.sum(jnp.where(eq & (idx < c), 1.0, 0.0),
                           axis=(0, 1), keepdims=True)

        def tie_body(i, t):
            b0 = jnp.int32(1) << (13 - 2 * i)
            b1 = b0 + b0
            c1, c2, c3 = t + b0, t + b1, t + b1 + b0
            n1, n2, n3 = count_lt(c1), count_lt(c2), count_lt(c3)
            return jnp.where(n3 < e, c3,
                             jnp.where(n2 < e, c2,
                                       jnp.where(n1 < e, c1, t)))

        i_star = jax.lax.fori_loop(0, 7, tie_body,
                                   jnp.zeros((1, 1), jnp.int32), unroll=7)
        i_last = i_star + jnp.int32(1)
        i_star = jnp.where(count_lt(i_last) < e, i_last, i_star)
        hard = gt | (eq & (idx <= i_star))
        pert_ref[...] = jnp.where(hard, (1.0 - khot) + khot, 0.0)

    @pl.when(j == 0)
    def _wait_x_copy():
        pltpu.make_async_copy(x_hbm_ref, x_vmem_ref, dma_sem).wait()

    x = x_vmem_ref[:, pl.ds(j * _C, _C)]
    pv = pv_ref[...]
    pert = pert_ref[pl.ds(j, 1), :]
    out_ref[...] = x + (pv - x) * pert


def kernel(x, logits, pert_val):
    return pl.pallas_call(
        _pert_kernel,
        grid=(_R,),
        in_specs=[
            pl.BlockSpec((_R, _C), lambda j: (0, 0)),
            pl.BlockSpec(memory_space=pl.ANY),
            pl.BlockSpec((1, _C), lambda j: (0, j)),
        ],
        out_specs=pl.BlockSpec((_B, _C), lambda j: (0, j)),
        out_shape=jax.ShapeDtypeStruct((_B, _P), jnp.float32),
        scratch_shapes=[pltpu.VMEM((_R, _C), jnp.float32),
                        pltpu.VMEM((_B, _P), jnp.float32),
                        pltpu.SemaphoreType.DMA],
        compiler_params=pltpu.CompilerParams(
            dimension_semantics=("arbitrary",)),
    )(logits.reshape(_R, _C), x, pert_val.reshape(1, _P))


# speedup vs baseline: 2.6239x; 1.0613x over previous
import jax
import jax.numpy as jnp
import numpy as np
from jax.experimental import pallas as pl
from jax.experimental.pallas import tpu as pltpu

_K = 64
_P = 32768
_B = 64
_R = 8
_C = _P // _R
_EPS = float(np.finfo(np.float32).tiny)


def _pert_kernel(logits_ref, x_hbm_ref, pv_ref, out_ref, pert_ref,
                 x_vmem_ref, dma_sem):
    j = pl.program_id(0)

    @pl.when(j == 0)
    def _start_x_copy():
        pltpu.make_async_copy(x_hbm_ref, x_vmem_ref, dma_sem).start()

    @pl.when(j == 0)
    def _compute_pert():
        lg = logits_ref[...]
        u0 = jnp.exp(lg - jnp.max(lg, axis=(0, 1), keepdims=True))

        def khot_body(_, carry):
            u, khot = carry
            s = jnp.sum(u, axis=(0, 1), keepdims=True)
            oa = u * (1.0 / s)
            khot = khot + oa
            u = u * jnp.maximum(1.0 - oa, _EPS)
            return (u, khot)

        zeros = jnp.zeros_like(u0)
        _, khot = jax.lax.fori_loop(0, _K, khot_body, (u0, zeros), unroll=16)

        kb = jax.lax.bitcast_convert_type(khot, jnp.int32)

        def count_ge(c):
            return jnp.sum(jnp.where(kb >= c, 1.0, 0.0),
                           axis=(0, 1), keepdims=True)

        def bit_body(i, t):
            b0 = jnp.int32(1) << (29 - 2 * i)
            b1 = b0 + b0
            c1, c2, c3 = t | b0, t | b1, t | (b1 | b0)
            n1, n2, n3 = count_ge(c1), count_ge(c2), count_ge(c3)
            return jnp.where(n3 >= 64.0, c3,
                             jnp.where(n2 >= 64.0, c2,
                                       jnp.where(n1 >= 64.0, c1, t)))

        t64 = jax.lax.fori_loop(0, 15, bit_body, jnp.zeros((1, 1), jnp.int32),
                                unroll=8)
        c_last = t64 | jnp.int32(1)
        t64 = jnp.where(count_ge(c_last) >= 64.0, c_last, t64)

        gt = kb > t64
        eq = kb == t64
        m = jnp.sum(jnp.where(gt, 1.0, 0.0), axis=(0, 1), keepdims=True)
        e = 64.0 - m
        idx = (jax.lax.broadcasted_iota(jnp.int32, (_R, _C), 0) * _C
               + jax.lax.broadcasted_iota(jnp.int32, (_R, _C), 1))

        def count_lt(c):
            return jnp.sum(jnp.where(eq & (idx < c), 1.0, 0.0),
                           axis=(0, 1), keepdims=True)

        def tie_body(i, t):
            b0 = jnp.int32(1) << (13 - 2 * i)
            b1 = b0 + b0
            c1, c2, c3 = t + b0, t + b1, t + b1 + b0
            n1, n2, n3 = count_lt(c1), count_lt(c2), count_lt(c3)
            return jnp.where(n3 < e, c3,
                             jnp.where(n2 < e, c2,
                                       jnp.where(n1 < e, c1, t)))

        def tie_search(_):
            t = jax.lax.fori_loop(0, 7, tie_body,
                                  jnp.zeros((1, 1), jnp.int32), unroll=7)
            c = t + jnp.int32(1)
            return jnp.where(count_lt(c) < e, c, t)

        n_ge = count_ge(t64)
        i_star = jax.lax.cond(jnp.all(n_ge > 64.0), tie_search,
                              lambda _: jnp.full((1, 1), _P, jnp.int32), None)
        hard = gt | (eq & (idx <= i_star))
        pert_ref[...] = jnp.where(hard, (1.0 - khot) + khot, 0.0)

    @pl.when(j == 0)
    def _wait_x_copy():
        pltpu.make_async_copy(x_hbm_ref, x_vmem_ref, dma_sem).wait()

    x = x_vmem_ref[:, pl.ds(j * _C, _C)]
    pv = pv_ref[...]
    pert = pert_ref[pl.ds(j, 1), :]
    out_ref[...] = x + (pv - x) * pert


def kernel(x, logits, pert_val):
    return pl.pallas_call(
        _pert_kernel,
        grid=(_R,),
        in_specs=[
            pl.BlockSpec((_R, _C), lambda j: (0, 0)),
            pl.BlockSpec(memory_space=pl.ANY),
            pl.BlockSpec((1, _C), lambda j: (0, j)),
        ],
        out_specs=pl.BlockSpec((_B, _C), lambda j: (0, j)),
        out_shape=jax.ShapeDtypeStruct((_B, _P), jnp.float32),
        scratch_shapes=[pltpu.VMEM((_R, _C), jnp.float32),
                        pltpu.VMEM((_B, _P), jnp.float32),
                        pltpu.SemaphoreType.DMA],
        compiler_params=pltpu.CompilerParams(
            dimension_semantics=("arbitrary",)),
    )(logits.reshape(_R, _C), x, pert_val.reshape(1, _P))


# speedup vs baseline: 2.6285x; 1.0018x over previous
import jax
import jax.numpy as jnp
import numpy as np
from jax.experimental import pallas as pl
from jax.experimental.pallas import tpu as pltpu

_K = 64
_P = 32768
_B = 64
_R = 8
_C = _P // _R
_EPS = float(np.finfo(np.float32).tiny)


def _pert_kernel(logits_ref, x_hbm_ref, pv_ref, out_ref, pert_ref,
                 x_vmem_ref, dma_sem):
    j = pl.program_id(0)

    @pl.when(j == 0)
    def _start_x_copy():
        pltpu.make_async_copy(x_hbm_ref, x_vmem_ref, dma_sem).start()

    @pl.when(j == 0)
    def _compute_pert():
        lg = logits_ref[...]
        u0 = jnp.exp(lg - jnp.max(lg, axis=(0, 1), keepdims=True))

        def khot_body(_, carry):
            u, khot = carry
            s = jnp.sum(u, axis=(0, 1), keepdims=True)
            oa = u * (1.0 / s)
            khot = khot + oa
            u = u * jnp.maximum(1.0 - oa, _EPS)
            return (u, khot)

        zeros = jnp.zeros_like(u0)
        _, khot = jax.lax.fori_loop(0, _K, khot_body, (u0, zeros), unroll=32)

        kb = jax.lax.bitcast_convert_type(khot, jnp.int32)

        def count_ge(c):
            return jnp.sum(jnp.where(kb >= c, 1.0, 0.0),
                           axis=(0, 1), keepdims=True)

        def bit_body(i, t):
            b0 = jnp.int32(1) << (29 - 2 * i)
            b1 = b0 + b0
            c1, c2, c3 = t | b0, t | b1, t | (b1 | b0)
            n1, n2, n3 = count_ge(c1), count_ge(c2), count_ge(c3)
            return jnp.where(n3 >= 64.0, c3,
                             jnp.where(n2 >= 64.0, c2,
                                       jnp.where(n1 >= 64.0, c1, t)))

        t64 = jax.lax.fori_loop(0, 15, bit_body, jnp.zeros((1, 1), jnp.int32),
                                unroll=8)
        c_last = t64 | jnp.int32(1)
        t64 = jnp.where(count_ge(c_last) >= 64.0, c_last, t64)

        gt = kb > t64
        eq = kb == t64
        m = jnp.sum(jnp.where(gt, 1.0, 0.0), axis=(0, 1), keepdims=True)
        e = 64.0 - m
        idx = (jax.lax.broadcasted_iota(jnp.int32, (_R, _C), 0) * _C
               + jax.lax.broadcasted_iota(jnp.int32, (_R, _C), 1))

        def count_lt(c):
            return jnp.sum(jnp.where(eq & (idx < c), 1.0, 0.0),
                           axis=(0, 1), keepdims=True)

        def tie_body(i, t):
            b0 = jnp.int32(1) << (13 - 2 * i)
            b1 = b0 + b0
            c1, c2, c3 = t + b0, t + b1, t + b1 + b0
            n1, n2, n3 = count_lt(c1), count_lt(c2), count_lt(c3)
            return jnp.where(n3 < e, c3,
                             jnp.where(n2 < e, c2,
                                       jnp.where(n1 < e, c1, t)))

        def tie_search(_):
            t = jax.lax.fori_loop(0, 7, tie_body,
                                  jnp.zeros((1, 1), jnp.int32), unroll=7)
            c = t + jnp.int32(1)
            return jnp.where(count_lt(c) < e, c, t)

        n_ge = count_ge(t64)
        i_star = jax.lax.cond(jnp.all(n_ge > 64.0), tie_search,
                              lambda _: jnp.full((1, 1), _P, jnp.int32), None)
        hard = gt | (eq & (idx <= i_star))
        pert_ref[...] = jnp.where(hard, (1.0 - khot) + khot, 0.0)

    @pl.when(j == 0)
    def _wait_x_copy():
        pltpu.make_async_copy(x_hbm_ref, x_vmem_ref, dma_sem).wait()

    x = x_vmem_ref[:, pl.ds(j * _C, _C)]
    pv = pv_ref[...]
    pert = pert_ref[pl.ds(j, 1), :]
    out_ref[...] = x + (pv - x) * pert


def kernel(x, logits, pert_val):
    return pl.pallas_call(
        _pert_kernel,
        grid=(_R,),
        in_specs=[
            pl.BlockSpec((_R, _C), lambda j: (0, 0)),
            pl.BlockSpec(memory_space=pl.ANY),
            pl.BlockSpec((1, _C), lambda j: (0, j)),
        ],
        out_specs=pl.BlockSpec((_B, _C), lambda j: (0, j)),
        out_shape=jax.ShapeDtypeStruct((_B, _P), jnp.float32),
        scratch_shapes=[pltpu.VMEM((_R, _C), jnp.float32),
                        pltpu.VMEM((_B, _P), jnp.float32),
                        pltpu.SemaphoreType.DMA],
        compiler_params=pltpu.CompilerParams(
            dimension_semantics=("arbitrary",)),
    )(logits.reshape(_R, _C), x, pert_val.reshape(1, _P))
